# baseline (speedup 1.0000x reference)
import functools

import jax
import jax.numpy as jnp
from jax import lax
from jax.experimental import pallas as pl
from jax.experimental.pallas import tpu as pltpu
from jax.experimental.pallas import tpu_sc as plsc

N = 10000
DEG = 16
D = 256
OUT = 256
R = 4
HID = 64
RESIDUAL = 0.12
EPS = 1e-5

NC = 2
NS = 16
NW = NC * NS
LANES = 16
CB = 32
NPW = 320
NBLK = NPW // CB
PADN = NW * NPW

BN = 400
NBLOCKS = N // BN


def _sc_body(h_hbm, g_hbm, src_hbm, rel_hbm, att_h_hbm, att_g_hbm,
             idx_v, rel_v, idxr, rows_h, rows_g, acc_h, acc_g, sem_h, sem_g):
  c = lax.axis_index("c")
  s = lax.axis_index("s")
  wid = s * NC + c
  wbase = wid * NPW

  def block_body(b, carry):
    base = wbase + b * CB
    pltpu.sync_copy(src_hbm.at[pl.ds(base, CB)], idx_v)
    pltpu.sync_copy(rel_hbm.at[pl.ds(base, CB)], rel_v)

    def node_body(i, carry2):
      n = base + i
      idxrow = idx_v[i, :]
      relrow = rel_v[i, :]
      for r in range(R):
        idxr[pl.ds(r * DEG, DEG)] = jnp.where(relrow == r, idxrow,
                                              jnp.int32(N))
      cp_h = pltpu.async_copy(h_hbm.at[idxr], rows_h, sem_h)
      cp_g = pltpu.async_copy(g_hbm.at[idxr], rows_g, sem_g)
      cp_h.wait()
      cp_g.wait()

      for r in range(R):
        for j in range(D // LANES):
          sl = pl.ds(j * LANES, LANES)
          ah = rows_h[r * DEG, sl]
          ag = rows_g[r * DEG, sl]
          for k in range(1, DEG):
            ah = jnp.maximum(ah, rows_h[r * DEG + k, sl])
            ag = jnp.maximum(ag, rows_g[r * DEG + k, sl])
          acc_h[r, sl] = ah
          acc_g[r, sl] = ag

      @pl.when(n < N)
      def _():
        pltpu.sync_copy(acc_h, att_h_hbm.at[n])
        pltpu.sync_copy(acc_g, att_g_hbm.at[n])
      return carry2

    lax.fori_loop(0, CB, node_body, 0)
    return carry

  lax.fori_loop(0, NBLK, block_body, 0)


@functools.cache
def _sc_gather_max():
  return pl.kernel(
    _sc_body,
    out_type=(jax.ShapeDtypeStruct((N, R, D), jnp.float32),
              jax.ShapeDtypeStruct((N, R, D), jnp.float32)),
    mesh=plsc.VectorSubcoreMesh(core_axis_name="c", subcore_axis_name="s",
                                num_cores=NC, num_subcores=NS),
    scratch_types=[
        pltpu.VMEM((CB, DEG), jnp.int32),
        pltpu.VMEM((CB, DEG), jnp.int32),
        pltpu.VMEM((R * DEG,), jnp.int32),
        pltpu.VMEM((R * DEG, D), jnp.float32),
        pltpu.VMEM((R * DEG, D), jnp.float32),
        pltpu.VMEM((R, D), jnp.float32),
        pltpu.VMEM((R, D), jnp.float32),
        pltpu.SemaphoreType.DMA,
        pltpu.SemaphoreType.DMA,
    ],
  )


def _leaky(x):
  return jnp.where(x >= 0, x, 0.2 * x)


def _dense1_body(att0_ref, att1_ref, h0_ref, h1_ref, wt0_ref, wt1_ref,
                 as0_ref, ad0_ref, as1_ref, ad1_ref, wma1_ref, bma1_ref,
                 wma2_ref, bma2_ref, raw_ref, stat_ref):
  i = pl.program_id(0)

  def edge_gat(att_ref, h_ref, wt_ref, a_src_ref, a_dst_ref):
    wt = wt_ref[...]
    hz = _leaky(jnp.dot(h_ref[...], wt, preferred_element_type=jnp.float32))
    ed = jnp.sum(hz * a_dst_ref[...], axis=1, keepdims=True)
    srcz = []
    ef = []
    for r in range(R):
      z = _leaky(jnp.dot(att_ref[:, r, :], wt,
                         preferred_element_type=jnp.float32))
      srcz.append(z)
      ef.append(_leaky(jnp.sum(z * a_src_ref[...], axis=1, keepdims=True) + ed))
    m = ef[0]
    for r in range(1, R):
      m = jnp.maximum(m, ef[r])
    ex = [jnp.exp(ef[r] - m) for r in range(R)]
    tot = ex[0]
    for r in range(1, R):
      tot = tot + ex[r]
    msg = ex[0] * srcz[0]
    for r in range(1, R):
      msg = msg + ex[r] * srcz[r]
    return msg / tot + RESIDUAL * hz

  msg0 = edge_gat(att0_ref, h0_ref, wt0_ref, as0_ref, ad0_ref)
  msg1 = edge_gat(att1_ref, h1_ref, wt1_ref, as1_ref, ad1_ref)

  wma1 = wma1_ref[...]
  bma1 = bma1_ref[...]
  wma2 = wma2_ref[...]
  bma2 = bma2_ref[...]

  def chan_score(msg):
    t = jnp.tanh(jnp.dot(msg, wma1, preferred_element_type=jnp.float32) + bma1)
    return jnp.tanh(jnp.sum(t * wma2, axis=1, keepdims=True) + bma2)

  w0 = chan_score(msg0)
  w1 = chan_score(msg1)
  m = jnp.maximum(w0, w1)
  e0 = jnp.exp(w0 - m)
  e1 = jnp.exp(w1 - m)
  multi = (e0 * msg0 + e1 * msg1) / (e0 + e1)
  raw_ref[...] = multi

  @pl.when(i == 0)
  def _():
    stat_ref[...] = jnp.zeros_like(stat_ref)
  ps = jnp.sum(multi, axis=0, keepdims=True)
  pss = jnp.sum(multi * multi, axis=0, keepdims=True)
  stat_ref[0:1, :] += ps
  stat_ref[1:2, :] += pss


def _dense2_body(raw_ref, stat_ref, gamma_ref, beta_ref, out_ref):
  s = stat_ref[0:1, :]
  ss = stat_ref[1:2, :]
  mean = s * (1.0 / N)
  var = ss * (1.0 / N) - mean * mean
  inv = lax.rsqrt(var + EPS)
  out_ref[...] = (raw_ref[...] - mean) * inv * gamma_ref[...] + beta_ref[...]


def _dense_stage(att0, att1, h, h_img, W_fc0, W_attn0, W_fc1, W_attn1,
                 W_ma1, b_ma1, W_ma2, b_ma2, gamma, beta):
  wt0 = W_fc0.T
  wt1 = W_fc1.T
  as0 = W_attn0[:, :OUT]
  ad0 = W_attn0[:, OUT:]
  as1 = W_attn1[:, :OUT]
  ad1 = W_attn1[:, OUT:]
  wma1t = W_ma1.T
  bma1 = b_ma1.reshape(1, HID)
  wma2 = W_ma2.reshape(1, HID)
  bma2 = b_ma2.reshape(1, 1)
  g2 = gamma.reshape(1, OUT)
  b2 = beta.reshape(1, OUT)

  full = lambda i: (0, 0)
  raw, stat = pl.pallas_call(
      _dense1_body,
      grid=(NBLOCKS,),
      in_specs=[
          pl.BlockSpec((BN, R, D), lambda i: (i, 0, 0)),
          pl.BlockSpec((BN, R, D), lambda i: (i, 0, 0)),
          pl.BlockSpec((BN, D), lambda i: (i, 0)),
          pl.BlockSpec((BN, D), lambda i: (i, 0)),
          pl.BlockSpec((D, OUT), full),
          pl.BlockSpec((D, OUT), full),
          pl.BlockSpec((1, OUT), full),
          pl.BlockSpec((1, OUT), full),
          pl.BlockSpec((1, OUT), full),
          pl.BlockSpec((1, OUT), full),
          pl.BlockSpec((OUT, HID), full),
          pl.BlockSpec((1, HID), full),
          pl.BlockSpec((1, HID), full),
          pl.BlockSpec((1, 1), full),
      ],
      out_specs=[
          pl.BlockSpec((BN, OUT), lambda i: (i, 0)),
          pl.BlockSpec((8, OUT), full),
      ],
      out_shape=[
          jax.ShapeDtypeStruct((N, OUT), jnp.float32),
          jax.ShapeDtypeStruct((8, OUT), jnp.float32),
      ],
      compiler_params=pltpu.CompilerParams(
          dimension_semantics=("arbitrary",)),
  )(att0, att1, h, h_img, wt0, wt1, as0, ad0, as1, ad1, wma1t, bma1,
    wma2, bma2)

  out = pl.pallas_call(
      _dense2_body,
      grid=(NBLOCKS,),
      in_specs=[
          pl.BlockSpec((BN, OUT), lambda i: (i, 0)),
          pl.BlockSpec((8, OUT), full),
          pl.BlockSpec((1, OUT), full),
          pl.BlockSpec((1, OUT), full),
      ],
      out_specs=pl.BlockSpec((BN, OUT), lambda i: (i, 0)),
      out_shape=jax.ShapeDtypeStruct((N, OUT), jnp.float32),
      compiler_params=pltpu.CompilerParams(
          dimension_semantics=("arbitrary",)),
  )(raw, stat, g2, b2)
  return out


def kernel(h, h_img, W_fc0, W_attn0, W_fc1, W_attn1, W_ma1, b_ma1, W_ma2,
           b_ma2, gamma, beta, edge_index, rel_type):
  src = edge_index[0]
  src2 = jnp.transpose(src.reshape(DEG, N)).astype(jnp.int32)
  rel2 = jnp.transpose(rel_type.reshape(DEG, N)).astype(jnp.int32)
  src2 = jnp.pad(src2, ((0, PADN - N), (0, 0)))
  rel2 = jnp.pad(rel2, ((0, PADN - N), (0, 0)))
  hp = jnp.concatenate([h, jnp.zeros((1, D), jnp.float32)], axis=0)
  gp = jnp.concatenate([h_img, jnp.zeros((1, D), jnp.float32)], axis=0)

  att0, att1 = _sc_gather_max()(hp, gp, src2, rel2)
  return _dense_stage(att0, att1, h, h_img, W_fc0, W_attn0, W_fc1, W_attn1,
                      W_ma1, b_ma1, W_ma2, b_ma2, gamma, beta)


# speedup vs baseline: 1.0004x; 1.0004x over previous
import functools

import jax
import jax.numpy as jnp
from jax import lax
from jax.experimental import pallas as pl
from jax.experimental.pallas import tpu as pltpu
from jax.experimental.pallas import tpu_sc as plsc

N = 10000
DEG = 16
D = 256
OUT = 256
R = 4
HID = 64
RESIDUAL = 0.12
EPS = 1e-5

NC = 2
NS = 16
NW = NC * NS
LANES = 16
CB = 32
NPW = 320
NBLK = NPW // CB
PADN = NW * NPW

BN = 400
NBLOCKS = N // BN


def _sc_body(h_hbm, g_hbm, src_hbm, rel_hbm, att_h_hbm, att_g_hbm,
             idx_v, rel_v, idxr, rows_h, rows_g, acc_h, acc_g,
             sems_h, sems_g, sems_oh, sems_og):
  c = lax.axis_index("c")
  s = lax.axis_index("s")
  wid = s * NC + c
  wbase = wid * NPW
  WROWS = NPW * DEG // 128
  rbase = wid * WROWS

  pltpu.sync_copy(src_hbm.at[pl.ds(rbase, WROWS)], idx_v)
  pltpu.sync_copy(rel_hbm.at[pl.ds(rbase, WROWS)], rel_v)

  def issue(i, slot):
    row = i // 8
    off = (i % 8) * DEG
    idxrow = idx_v[row, pl.ds(off, DEG)]
    relrow = rel_v[row, pl.ds(off, DEG)]
    for r in range(R):
      idxr[slot, r, :] = jnp.where(relrow == r, idxrow, jnp.int32(N))
    for r in range(R):
      pltpu.async_copy(h_hbm.at[idxr.at[slot, r]],
                       rows_h.at[slot, pl.ds(r * DEG, DEG)],
                       sems_h.at[slot])
      pltpu.async_copy(g_hbm.at[idxr.at[slot, r]],
                       rows_g.at[slot, pl.ds(r * DEG, DEG)],
                       sems_g.at[slot])

  def wait(i, slot):
    for r in range(R):
      pltpu.make_async_copy(h_hbm.at[idxr.at[slot, r]],
                            rows_h.at[slot, pl.ds(r * DEG, DEG)],
                            sems_h.at[slot]).wait()
      pltpu.make_async_copy(g_hbm.at[idxr.at[slot, r]],
                            rows_g.at[slot, pl.ds(r * DEG, DEG)],
                            sems_g.at[slot]).wait()

  def wait_out(n, slot):
    pltpu.make_async_copy(acc_h.at[slot], att_h_hbm.at[n],
                          sems_oh.at[slot]).wait()
    pltpu.make_async_copy(acc_g.at[slot], att_g_hbm.at[n],
                          sems_og.at[slot]).wait()

  issue(0, 0)
  issue(1, 1)

  def pair_body(p, carry):
    for par in range(2):
      i = p * 2 + par
      n = wbase + i
      wait(i, par)
      @pl.when(jnp.logical_and(i >= 2, n - 2 < N))
      def _():
        wait_out(n - 2, par)
      for r in range(R):
        for j in range(D // LANES):
          sl = pl.ds(j * LANES, LANES)
          ah = rows_h[par, r * DEG, sl]
          ag = rows_g[par, r * DEG, sl]
          for k in range(1, DEG):
            ah = jnp.maximum(ah, rows_h[par, r * DEG + k, sl])
            ag = jnp.maximum(ag, rows_g[par, r * DEG + k, sl])
          acc_h[par, r, sl] = ah
          acc_g[par, r, sl] = ag

      @pl.when(i + 2 < NPW)
      def _():
        issue(i + 2, par)

      @pl.when(n < N)
      def _():
        pltpu.async_copy(acc_h.at[par], att_h_hbm.at[n], sems_oh.at[par])
        pltpu.async_copy(acc_g.at[par], att_g_hbm.at[n], sems_og.at[par])
    return carry

  lax.fori_loop(0, NPW // 2, pair_body, 0)

  for par in range(2):
    n = wbase + NPW - 2 + par
    @pl.when(n < N)
    def _(n=n, par=par):
      wait_out(n, par)


@functools.cache
def _sc_gather_max():
  return pl.kernel(
    _sc_body,
    out_type=(jax.ShapeDtypeStruct((N, R, D), jnp.float32),
              jax.ShapeDtypeStruct((N, R, D), jnp.float32)),
    mesh=plsc.VectorSubcoreMesh(core_axis_name="c", subcore_axis_name="s",
                                num_cores=NC, num_subcores=NS),
    scratch_types=[
        pltpu.VMEM((NPW * DEG // 128, 128), jnp.int32),
        pltpu.VMEM((NPW * DEG // 128, 128), jnp.int32),
        pltpu.VMEM((2, R, DEG), jnp.int32),
        pltpu.VMEM((2, R * DEG, D), jnp.float32),
        pltpu.VMEM((2, R * DEG, D), jnp.float32),
        pltpu.VMEM((2, R, D), jnp.float32),
        pltpu.VMEM((2, R, D), jnp.float32),
        pltpu.SemaphoreType.DMA((2,)),
        pltpu.SemaphoreType.DMA((2,)),
        pltpu.SemaphoreType.DMA((2,)),
        pltpu.SemaphoreType.DMA((2,)),
    ],
  )


def _leaky(x):
  return jnp.where(x >= 0, x, 0.2 * x)


def _dense1_body(att0_ref, att1_ref, h0_ref, h1_ref, wt0_ref, wt1_ref,
                 as0_ref, ad0_ref, as1_ref, ad1_ref, wma1_ref, bma1_ref,
                 wma2_ref, bma2_ref, raw_ref, stat_ref):
  i = pl.program_id(0)

  def edge_gat(att_ref, h_ref, wt_ref, a_src_ref, a_dst_ref):
    wt = wt_ref[...]
    hz = _leaky(jnp.dot(h_ref[...], wt, preferred_element_type=jnp.float32))
    ed = jnp.sum(hz * a_dst_ref[...], axis=1, keepdims=True)
    srcz = []
    ef = []
    for r in range(R):
      z = _leaky(jnp.dot(att_ref[:, r, :], wt,
                         preferred_element_type=jnp.float32))
      srcz.append(z)
      ef.append(_leaky(jnp.sum(z * a_src_ref[...], axis=1, keepdims=True) + ed))
    m = ef[0]
    for r in range(1, R):
      m = jnp.maximum(m, ef[r])
    ex = [jnp.exp(ef[r] - m) for r in range(R)]
    tot = ex[0]
    for r in range(1, R):
      tot = tot + ex[r]
    msg = ex[0] * srcz[0]
    for r in range(1, R):
      msg = msg + ex[r] * srcz[r]
    return msg / tot + RESIDUAL * hz

  msg0 = edge_gat(att0_ref, h0_ref, wt0_ref, as0_ref, ad0_ref)
  msg1 = edge_gat(att1_ref, h1_ref, wt1_ref, as1_ref, ad1_ref)

  wma1 = wma1_ref[...]
  bma1 = bma1_ref[...]
  wma2 = wma2_ref[...]
  bma2 = bma2_ref[...]

  def chan_score(msg):
    t = jnp.tanh(jnp.dot(msg, wma1, preferred_element_type=jnp.float32) + bma1)
    return jnp.tanh(jnp.sum(t * wma2, axis=1, keepdims=True) + bma2)

  w0 = chan_score(msg0)
  w1 = chan_score(msg1)
  m = jnp.maximum(w0, w1)
  e0 = jnp.exp(w0 - m)
  e1 = jnp.exp(w1 - m)
  multi = (e0 * msg0 + e1 * msg1) / (e0 + e1)
  raw_ref[...] = multi

  @pl.when(i == 0)
  def _():
    stat_ref[...] = jnp.zeros_like(stat_ref)
  ps = jnp.sum(multi, axis=0, keepdims=True)
  pss = jnp.sum(multi * multi, axis=0, keepdims=True)
  stat_ref[0:1, :] += ps
  stat_ref[1:2, :] += pss


def _dense2_body(raw_ref, stat_ref, gamma_ref, beta_ref, out_ref):
  s = stat_ref[0:1, :]
  ss = stat_ref[1:2, :]
  mean = s * (1.0 / N)
  var = ss * (1.0 / N) - mean * mean
  inv = lax.rsqrt(var + EPS)
  out_ref[...] = (raw_ref[...] - mean) * inv * gamma_ref[...] + beta_ref[...]


def _dense_stage(att0, att1, h, h_img, W_fc0, W_attn0, W_fc1, W_attn1,
                 W_ma1, b_ma1, W_ma2, b_ma2, gamma, beta):
  wt0 = W_fc0.T
  wt1 = W_fc1.T
  as0 = W_attn0[:, :OUT]
  ad0 = W_attn0[:, OUT:]
  as1 = W_attn1[:, :OUT]
  ad1 = W_attn1[:, OUT:]
  wma1t = W_ma1.T
  bma1 = b_ma1.reshape(1, HID)
  wma2 = W_ma2.reshape(1, HID)
  bma2 = b_ma2.reshape(1, 1)
  g2 = gamma.reshape(1, OUT)
  b2 = beta.reshape(1, OUT)

  full = lambda i: (0, 0)
  raw, stat = pl.pallas_call(
      _dense1_body,
      grid=(NBLOCKS,),
      in_specs=[
          pl.BlockSpec((BN, R, D), lambda i: (i, 0, 0)),
          pl.BlockSpec((BN, R, D), lambda i: (i, 0, 0)),
          pl.BlockSpec((BN, D), lambda i: (i, 0)),
          pl.BlockSpec((BN, D), lambda i: (i, 0)),
          pl.BlockSpec((D, OUT), full),
          pl.BlockSpec((D, OUT), full),
          pl.BlockSpec((1, OUT), full),
          pl.BlockSpec((1, OUT), full),
          pl.BlockSpec((1, OUT), full),
          pl.BlockSpec((1, OUT), full),
          pl.BlockSpec((OUT, HID), full),
          pl.BlockSpec((1, HID), full),
          pl.BlockSpec((1, HID), full),
          pl.BlockSpec((1, 1), full),
      ],
      out_specs=[
          pl.BlockSpec((BN, OUT), lambda i: (i, 0)),
          pl.BlockSpec((8, OUT), full),
      ],
      out_shape=[
          jax.ShapeDtypeStruct((N, OUT), jnp.float32),
          jax.ShapeDtypeStruct((8, OUT), jnp.float32),
      ],
      compiler_params=pltpu.CompilerParams(
          dimension_semantics=("arbitrary",)),
  )(att0, att1, h, h_img, wt0, wt1, as0, ad0, as1, ad1, wma1t, bma1,
    wma2, bma2)

  out = pl.pallas_call(
      _dense2_body,
      grid=(NBLOCKS,),
      in_specs=[
          pl.BlockSpec((BN, OUT), lambda i: (i, 0)),
          pl.BlockSpec((8, OUT), full),
          pl.BlockSpec((1, OUT), full),
          pl.BlockSpec((1, OUT), full),
      ],
      out_specs=pl.BlockSpec((BN, OUT), lambda i: (i, 0)),
      out_shape=jax.ShapeDtypeStruct((N, OUT), jnp.float32),
      compiler_params=pltpu.CompilerParams(
          dimension_semantics=("arbitrary",)),
  )(raw, stat, g2, b2)
  return out


def kernel(h, h_img, W_fc0, W_attn0, W_fc1, W_attn1, W_ma1, b_ma1, W_ma2,
           b_ma2, gamma, beta, edge_index, rel_type):
  src = edge_index[0]
  src2 = jnp.transpose(src.reshape(DEG, N)).astype(jnp.int32)
  rel2 = jnp.transpose(rel_type.reshape(DEG, N)).astype(jnp.int32)
  src2 = jnp.pad(src2, ((0, PADN - N), (0, 0))).reshape(-1, 128)
  rel2 = jnp.pad(rel2, ((0, PADN - N), (0, 0))).reshape(-1, 128)
  hp = jnp.concatenate([h, jnp.zeros((1, D), jnp.float32)], axis=0)
  gp = jnp.concatenate([h_img, jnp.zeros((1, D), jnp.float32)], axis=0)

  att0, att1 = _sc_gather_max()(hp, gp, src2, rel2)
  return _dense_stage(att0, att1, h, h_img, W_fc0, W_attn0, W_fc1, W_attn1,
                      W_ma1, b_ma1, W_ma2, b_ma2, gamma, beta)
